# baseline (speedup 1.0000x reference)
import jax
import jax.numpy as jnp
from jax import lax
from jax.experimental import pallas as pl
from jax.experimental.pallas import tpu as pltpu


def _round_up(x, m):
    return (x + m - 1) // m * m


def _pad_to(a, shape):
    pads = [(0, s - d) for d, s in zip(a.shape, shape)]
    if all(p == (0, 0) for p in pads):
        return a
    return jnp.pad(a, pads)


def _make_fused_body(total_t, ch, n_b, bp, hp):
    def body(idx_ref, emb_hbm_ref, wih_ref, brnn_ref, whh_ref, wfc_ref,
             bfc_ref, out_ref, h_ref, tab_ref, xa_ref, xb_ref, xp_ref,
             tab_sem):
        c = pl.program_id(0)
        nsteps = pl.num_programs(0)

        @pl.when(c == 0)
        def _():
            h_ref[...] = jnp.zeros_like(h_ref)
            cp = pltpu.make_async_copy(emb_hbm_ref, tab_ref.at[:, 0, :],
                                       tab_sem)
            cp.start()
            cp.wait()

        hb = bp // 2

        def phase(gat_ref, con_ref):
            base_flat = jnp.minimum(c * ch, total_t - ch) * n_b

            for t in range(ch):
                for b in range(n_b):
                    i = idx_ref[base_flat + (t * n_b + b)]
                    gat_ref[t, b] = tab_ref[i, 0]
            brnn = brnn_ref[...]
            valid = c > 0
            for t in range(ch):
                xp_ref[t] = jnp.where(
                    valid,
                    jnp.dot(con_ref[t].astype(jnp.bfloat16), wih_ref[...],
                            preferred_element_type=jnp.float32) + brnn,
                    0.0)
            ha = h_ref[0:hb]
            hc = h_ref[hb:]
            for t in range(ch):
                pre_a = xp_ref[t, 0:hb] + jnp.dot(
                    ha.astype(jnp.bfloat16), whh_ref[...],
                    preferred_element_type=jnp.float32)
                pre_c = xp_ref[t, hb:] + jnp.dot(
                    hc.astype(jnp.bfloat16), whh_ref[...],
                    preferred_element_type=jnp.float32)
                ha = jnp.tanh(pre_a)
                hc = jnp.tanh(pre_c)
            h_ref[0:hb] = ha
            h_ref[hb:] = hc

        @pl.when(lax.rem(c, 2) == 0)
        def _():
            phase(xa_ref, xb_ref)

        @pl.when(lax.rem(c, 2) == 1)
        def _():
            phase(xb_ref, xa_ref)

        @pl.when(c == nsteps - 1)
        def _():
            out_ref[...] = (
                jnp.dot(h_ref[...].astype(jnp.bfloat16), wfc_ref[...],
                        preferred_element_type=jnp.float32) + bfc_ref[...]
            ).astype(out_ref.dtype)

    return body


def _fused_kernel(x_idx, embedding, w_ih_t, w_hh_t, b_rnn, w_fc_t, b_fc, ch):
    B, T = x_idx.shape
    V, H = embedding.shape
    O = w_fc_t.shape[1]
    Hp = _round_up(H, 128)
    Op = _round_up(O, 128)
    Bp = _round_up(B, 8)
    Vp = _round_up(V, 8)

    emb = _pad_to(embedding, (Vp, Hp))
    wih = _pad_to(w_ih_t, (Hp, Hp)).astype(jnp.bfloat16)
    brnn = _pad_to(b_rnn, (1, Hp))
    whh = _pad_to(w_hh_t, (Hp, Hp)).astype(jnp.bfloat16)
    wfc = _pad_to(w_fc_t, (Hp, Op)).astype(jnp.bfloat16)
    bfc = _pad_to(b_fc, (1, Op))

    n_chunks = T // ch
    body = _make_fused_body(T, ch, B, Bp, Hp)

    grid_spec = pltpu.PrefetchScalarGridSpec(
        num_scalar_prefetch=1,
        grid=(n_chunks + 1,),
        in_specs=[
            pl.BlockSpec(memory_space=pl.ANY),
            pl.BlockSpec((Hp, Hp), lambda c, idx: (0, 0)),
            pl.BlockSpec((1, Hp), lambda c, idx: (0, 0)),
            pl.BlockSpec((Hp, Hp), lambda c, idx: (0, 0)),
            pl.BlockSpec((Hp, Op), lambda c, idx: (0, 0)),
            pl.BlockSpec((1, Op), lambda c, idx: (0, 0)),
        ],
        out_specs=pl.BlockSpec((Bp, Op), lambda c, idx: (0, 0)),
        scratch_shapes=[
            pltpu.VMEM((Bp, Hp), jnp.float32),
            pltpu.VMEM((Vp, 1, Hp), jnp.float32),
            pltpu.VMEM((ch, Bp, Hp), jnp.float32),
            pltpu.VMEM((ch, Bp, Hp), jnp.float32),
            pltpu.VMEM((ch, Bp, Hp), jnp.float32),
            pltpu.SemaphoreType.DMA,
        ],
    )
    out_p = pl.pallas_call(
        body,
        out_shape=jax.ShapeDtypeStruct((Bp, Op), jnp.float32),
        grid_spec=grid_spec,
        compiler_params=pltpu.CompilerParams(
            dimension_semantics=("arbitrary",),
            vmem_limit_bytes=63 * (1 << 20),
        ),
    )(x_idx.T.reshape(-1), emb, wih, brnn, whh, wfc, bfc)
    return out_p[:B, :O]


def _make_chunked_body(total_t, chunk, tb, hp, needs_mask):
    def body(emb_ref, wih_ref, brnn_ref, whh_ref, wfc_ref, bfc_ref,
             out_ref, h_ref, xp_ref):
        c = pl.program_id(0)

        @pl.when(c == 0)
        def _():
            h_ref[...] = jnp.zeros_like(h_ref)

        wih = wih_ref[...]
        brnn = brnn_ref[...]
        for g in range(chunk):
            xp_ref[g] = (jnp.dot(emb_ref[g], wih,
                                 preferred_element_type=jnp.float32) + brnn)

        whh = whh_ref[...]
        base = c * chunk
        h = h_ref[...]
        for t in range(chunk):
            pre = xp_ref[t] + jnp.dot(h, whh,
                                      preferred_element_type=jnp.float32)
            h_new = jnp.tanh(pre)
            if needs_mask:
                h_new = jnp.where(base + t < total_t, h_new, h)
            h = h_new
        h_ref[...] = h

        @pl.when(c == pl.num_programs(0) - 1)
        def _():
            out_ref[...] = (
                jnp.dot(h, wfc_ref[...], preferred_element_type=jnp.float32)
                + bfc_ref[...]
            ).astype(out_ref.dtype)

    return body


def _chunked_kernel(x_idx, embedding, w_ih_t, w_hh_t, b_rnn, w_fc_t, b_fc):
    B, T = x_idx.shape
    H = embedding.shape[1]
    O = w_fc_t.shape[1]
    Hp = _round_up(H, 128)
    Op = _round_up(O, 128)
    Bp = _round_up(B, 8)

    chunk = 0
    for cand in range(min(T, 32), 0, -1):
        if T % cand == 0:
            chunk = cand
            break
    if chunk < 8 and T > 32:
        chunk = 32
    n_chunks = -(-T // chunk)
    Tp = n_chunks * chunk
    needs_mask = Tp != T

    emb_tb = embedding[x_idx.T].astype(jnp.bfloat16)
    emb_tb = _pad_to(emb_tb, (Tp, Bp, Hp))
    wih = _pad_to(w_ih_t, (H, Hp)).astype(jnp.bfloat16)
    wih = _pad_to(wih, (Hp, Hp))
    brnn = _pad_to(b_rnn, (1, Hp))
    whh = _pad_to(w_hh_t, (Hp, Hp))
    wfc = _pad_to(w_fc_t, (Hp, Op))
    bfc = _pad_to(b_fc, (1, Op))

    body = _make_chunked_body(T, chunk, Bp, Hp, needs_mask)
    out_p = pl.pallas_call(
        body,
        grid=(n_chunks,),
        in_specs=[
            pl.BlockSpec((chunk, Bp, Hp), lambda c: (c, 0, 0)),
            pl.BlockSpec((Hp, Hp), lambda c: (0, 0)),
            pl.BlockSpec((1, Hp), lambda c: (0, 0)),
            pl.BlockSpec((Hp, Hp), lambda c: (0, 0)),
            pl.BlockSpec((Hp, Op), lambda c: (0, 0)),
            pl.BlockSpec((1, Op), lambda c: (0, 0)),
        ],
        out_specs=pl.BlockSpec((Bp, Op), lambda c: (0, 0)),
        out_shape=jax.ShapeDtypeStruct((Bp, Op), jnp.float32),
        scratch_shapes=[
            pltpu.VMEM((Bp, Hp), jnp.float32),
            pltpu.VMEM((chunk, Bp, Hp), jnp.float32),
        ],
        compiler_params=pltpu.CompilerParams(
            dimension_semantics=("arbitrary",),
            vmem_limit_bytes=100 * (1 << 20),
        ),
    )(emb_tb, wih, brnn, whh, wfc, bfc)
    return out_p[:B, :O]


def kernel(x_idx, embedding, w_ih_t, w_hh_t, b_rnn, w_fc_t, b_fc):
    B, T = x_idx.shape
    V, H = embedding.shape

    ch = 0
    for cand in (4, 2, 1):
        if T % cand == 0:
            ch = cand
            break

    Hp = _round_up(H, 128)
    Bp = _round_up(B, 8)
    Vp = _round_up(V, 8)
    vmem_bytes = (Vp * Hp + 3 * ch * Bp * Hp + 2 * Hp * Hp + Bp * Hp) * 4
    if (ch > 0 and B * ch <= 2048 and T >= 2 * ch
            and vmem_bytes <= 58 * (1 << 20)):
        return _fused_kernel(x_idx, embedding, w_ih_t, w_hh_t, b_rnn,
                             w_fc_t, b_fc, ch)
    return _chunked_kernel(x_idx, embedding, w_ih_t, w_hh_t, b_rnn,
                           w_fc_t, b_fc)


# speedup vs baseline: 1.1010x; 1.1010x over previous
import jax
import jax.numpy as jnp
from jax import lax
from jax.experimental import pallas as pl
from jax.experimental.pallas import tpu as pltpu


def _round_up(x, m):
    return (x + m - 1) // m * m


def _pad_to(a, shape):
    pads = [(0, s - d) for d, s in zip(a.shape, shape)]
    if all(p == (0, 0) for p in pads):
        return a
    return jnp.pad(a, pads)


def _make_fused_body(total_t, ch, n_b, bp, hp):
    def body(idx_ref, emb_hbm_ref, wih_ref, brnn_ref, whh_ref, wfc_ref,
             bfc_ref, out_ref, h_ref, tab_ref, xa_ref, xb_ref, xp_ref,
             tab_sem):
        c = pl.program_id(0)
        nsteps = pl.num_programs(0)

        @pl.when(c == 0)
        def _():
            h_ref[...] = jnp.zeros_like(h_ref)
            cp = pltpu.make_async_copy(emb_hbm_ref, tab_ref.at[:, 0, :],
                                       tab_sem)
            cp.start()
            cp.wait()

        def phase(gat_ref, con_ref):
            base = jnp.minimum(c * ch, total_t - ch)

            def gather_slice(t, lo, hi):
                for b in range(lo, hi):
                    i = idx_ref[base + t, b]
                    gat_ref[t, b] = tab_ref[i, 0]

            for t in range(ch):
                gather_slice(t, 0, n_b)
            brnn = brnn_ref[...]
            for t in range(ch):
                xp_ref[t] = (
                    jnp.dot(con_ref[t].astype(jnp.bfloat16), wih_ref[...],
                            preferred_element_type=jnp.float32) + brnn)
            hb = bp // 2
            ha = h_ref[0:hb]
            hc = h_ref[hb:]
            valid = c > 0
            for t in range(ch):
                pre_a = xp_ref[t, 0:hb] + jnp.dot(
                    ha.astype(jnp.bfloat16), whh_ref[...],
                    preferred_element_type=jnp.float32)
                pre_c = xp_ref[t, hb:] + jnp.dot(
                    hc.astype(jnp.bfloat16), whh_ref[...],
                    preferred_element_type=jnp.float32)
                ha = jnp.where(valid, jnp.tanh(pre_a), ha)
                hc = jnp.where(valid, jnp.tanh(pre_c), hc)
            h_ref[0:hb] = ha
            h_ref[hb:] = hc

        @pl.when(lax.rem(c, 2) == 0)
        def _():
            phase(xa_ref, xb_ref)

        @pl.when(lax.rem(c, 2) == 1)
        def _():
            phase(xb_ref, xa_ref)

        @pl.when(c == nsteps - 1)
        def _():
            out_ref[...] = (
                jnp.dot(h_ref[...].astype(jnp.bfloat16), wfc_ref[...],
                        preferred_element_type=jnp.float32) + bfc_ref[...]
            ).astype(out_ref.dtype)

    return body


def _fused_kernel(x_idx, embedding, w_ih_t, w_hh_t, b_rnn, w_fc_t, b_fc, ch):
    B, T = x_idx.shape
    V, H = embedding.shape
    O = w_fc_t.shape[1]
    Hp = _round_up(H, 128)
    Op = _round_up(O, 128)
    Bp = _round_up(B, 8)
    Vp = _round_up(V, 8)

    emb = _pad_to(embedding, (Vp, Hp))
    wih = _pad_to(w_ih_t, (Hp, Hp)).astype(jnp.bfloat16)
    brnn = _pad_to(b_rnn, (1, Hp))
    whh = _pad_to(w_hh_t, (Hp, Hp)).astype(jnp.bfloat16)
    wfc = _pad_to(w_fc_t, (Hp, Op)).astype(jnp.bfloat16)
    bfc = _pad_to(b_fc, (1, Op))

    n_chunks = T // ch
    body = _make_fused_body(T, ch, B, Bp, Hp)

    grid_spec = pltpu.PrefetchScalarGridSpec(
        num_scalar_prefetch=1,
        grid=(n_chunks + 1,),
        in_specs=[
            pl.BlockSpec(memory_space=pl.ANY),
            pl.BlockSpec((Hp, Hp), lambda c, idx: (0, 0)),
            pl.BlockSpec((1, Hp), lambda c, idx: (0, 0)),
            pl.BlockSpec((Hp, Hp), lambda c, idx: (0, 0)),
            pl.BlockSpec((Hp, Op), lambda c, idx: (0, 0)),
            pl.BlockSpec((1, Op), lambda c, idx: (0, 0)),
        ],
        out_specs=pl.BlockSpec((Bp, Op), lambda c, idx: (0, 0)),
        scratch_shapes=[
            pltpu.VMEM((Bp, Hp), jnp.float32),
            pltpu.VMEM((Vp, 1, Hp), jnp.float32),
            pltpu.VMEM((ch, Bp, Hp), jnp.float32),
            pltpu.VMEM((ch, Bp, Hp), jnp.float32),
            pltpu.VMEM((ch, Bp, Hp), jnp.float32),
            pltpu.SemaphoreType.DMA,
        ],
    )
    out_p = pl.pallas_call(
        body,
        out_shape=jax.ShapeDtypeStruct((Bp, Op), jnp.float32),
        grid_spec=grid_spec,
        compiler_params=pltpu.CompilerParams(
            dimension_semantics=("arbitrary",),
            vmem_limit_bytes=63 * (1 << 20),
        ),
    )(x_idx.T, emb, wih, brnn, whh, wfc, bfc)
    return out_p[:B, :O]


def _make_chunked_body(total_t, chunk, tb, hp, needs_mask):
    def body(emb_ref, wih_ref, brnn_ref, whh_ref, wfc_ref, bfc_ref,
             out_ref, h_ref, xp_ref):
        c = pl.program_id(0)

        @pl.when(c == 0)
        def _():
            h_ref[...] = jnp.zeros_like(h_ref)

        wih = wih_ref[...]
        brnn = brnn_ref[...]
        for g in range(chunk):
            xp_ref[g] = (jnp.dot(emb_ref[g], wih,
                                 preferred_element_type=jnp.float32) + brnn)

        whh = whh_ref[...]
        base = c * chunk
        h = h_ref[...]
        for t in range(chunk):
            pre = xp_ref[t] + jnp.dot(h, whh,
                                      preferred_element_type=jnp.float32)
            h_new = jnp.tanh(pre)
            if needs_mask:
                h_new = jnp.where(base + t < total_t, h_new, h)
            h = h_new
        h_ref[...] = h

        @pl.when(c == pl.num_programs(0) - 1)
        def _():
            out_ref[...] = (
                jnp.dot(h, wfc_ref[...], preferred_element_type=jnp.float32)
                + bfc_ref[...]
            ).astype(out_ref.dtype)

    return body


def _chunked_kernel(x_idx, embedding, w_ih_t, w_hh_t, b_rnn, w_fc_t, b_fc):
    B, T = x_idx.shape
    H = embedding.shape[1]
    O = w_fc_t.shape[1]
    Hp = _round_up(H, 128)
    Op = _round_up(O, 128)
    Bp = _round_up(B, 8)

    chunk = 0
    for cand in range(min(T, 32), 0, -1):
        if T % cand == 0:
            chunk = cand
            break
    if chunk < 8 and T > 32:
        chunk = 32
    n_chunks = -(-T // chunk)
    Tp = n_chunks * chunk
    needs_mask = Tp != T

    emb_tb = embedding[x_idx.T].astype(jnp.bfloat16)
    emb_tb = _pad_to(emb_tb, (Tp, Bp, Hp))
    wih = _pad_to(w_ih_t, (H, Hp)).astype(jnp.bfloat16)
    wih = _pad_to(wih, (Hp, Hp))
    brnn = _pad_to(b_rnn, (1, Hp))
    whh = _pad_to(w_hh_t, (Hp, Hp))
    wfc = _pad_to(w_fc_t, (Hp, Op))
    bfc = _pad_to(b_fc, (1, Op))

    body = _make_chunked_body(T, chunk, Bp, Hp, needs_mask)
    out_p = pl.pallas_call(
        body,
        grid=(n_chunks,),
        in_specs=[
            pl.BlockSpec((chunk, Bp, Hp), lambda c: (c, 0, 0)),
            pl.BlockSpec((Hp, Hp), lambda c: (0, 0)),
            pl.BlockSpec((1, Hp), lambda c: (0, 0)),
            pl.BlockSpec((Hp, Hp), lambda c: (0, 0)),
            pl.BlockSpec((Hp, Op), lambda c: (0, 0)),
            pl.BlockSpec((1, Op), lambda c: (0, 0)),
        ],
        out_specs=pl.BlockSpec((Bp, Op), lambda c: (0, 0)),
        out_shape=jax.ShapeDtypeStruct((Bp, Op), jnp.float32),
        scratch_shapes=[
            pltpu.VMEM((Bp, Hp), jnp.float32),
            pltpu.VMEM((chunk, Bp, Hp), jnp.float32),
        ],
        compiler_params=pltpu.CompilerParams(
            dimension_semantics=("arbitrary",),
            vmem_limit_bytes=100 * (1 << 20),
        ),
    )(emb_tb, wih, brnn, whh, wfc, bfc)
    return out_p[:B, :O]


def kernel(x_idx, embedding, w_ih_t, w_hh_t, b_rnn, w_fc_t, b_fc):
    B, T = x_idx.shape
    V, H = embedding.shape

    ch = 0
    for cand in (4, 2, 1):
        if T % cand == 0:
            ch = cand
            break

    Hp = _round_up(H, 128)
    Bp = _round_up(B, 8)
    Vp = _round_up(V, 8)
    vmem_bytes = (Vp * Hp + 3 * ch * Bp * Hp + 2 * Hp * Hp + Bp * Hp) * 4
    if (ch > 0 and B * ch <= 2048 and T >= 2 * ch
            and vmem_bytes <= 58 * (1 << 20)):
        return _fused_kernel(x_idx, embedding, w_ih_t, w_hh_t, b_rnn,
                             w_fc_t, b_fc, ch)
    return _chunked_kernel(x_idx, embedding, w_ih_t, w_hh_t, b_rnn,
                           w_fc_t, b_fc)


# speedup vs baseline: 1.1231x; 1.0201x over previous
import jax
import jax.numpy as jnp
from jax import lax
from jax.experimental import pallas as pl
from jax.experimental.pallas import tpu as pltpu


def _round_up(x, m):
    return (x + m - 1) // m * m


def _pad_to(a, shape):
    pads = [(0, s - d) for d, s in zip(a.shape, shape)]
    if all(p == (0, 0) for p in pads):
        return a
    return jnp.pad(a, pads)


def _make_fused_body(total_t, ch, n_b, bp, hp):
    def body(idx_ref, emb_hbm_ref, wih_ref, brnn_ref, whh_ref, wfc_ref,
             bfc_ref, out_ref, h_ref, tab_ref, xa_ref, xb_ref, xp_ref,
             tab_sem):
        c = pl.program_id(0)
        nsteps = pl.num_programs(0)

        @pl.when(c == 0)
        def _():
            h_ref[...] = jnp.zeros_like(h_ref)
            cp = pltpu.make_async_copy(emb_hbm_ref, tab_ref.at[:, 0, :],
                                       tab_sem)
            cp.start()
            cp.wait()

        def phase(gat_ref, con_ref):
            base_flat = jnp.minimum(c * ch, total_t - ch) * n_b

            def gather_slice(t, lo, hi):
                for b in range(lo, hi):
                    i = idx_ref[base_flat + (t * n_b + b)]
                    gat_ref[t, b] = tab_ref[i, 0]

            for t in range(ch):
                gather_slice(t, 0, n_b)
            brnn = brnn_ref[...]
            for t in range(ch):
                xp_ref[t] = (
                    jnp.dot(con_ref[t].astype(jnp.bfloat16), wih_ref[...],
                            preferred_element_type=jnp.float32) + brnn)
            hb = bp // 2
            ha = h_ref[0:hb]
            hc = h_ref[hb:]
            valid = c > 0
            for t in range(ch):
                pre_a = xp_ref[t, 0:hb] + jnp.dot(
                    ha.astype(jnp.bfloat16), whh_ref[...],
                    preferred_element_type=jnp.float32)
                pre_c = xp_ref[t, hb:] + jnp.dot(
                    hc.astype(jnp.bfloat16), whh_ref[...],
                    preferred_element_type=jnp.float32)
                ha = jnp.where(valid, jnp.tanh(pre_a), ha)
                hc = jnp.where(valid, jnp.tanh(pre_c), hc)
            h_ref[0:hb] = ha
            h_ref[hb:] = hc

        @pl.when(lax.rem(c, 2) == 0)
        def _():
            phase(xa_ref, xb_ref)

        @pl.when(lax.rem(c, 2) == 1)
        def _():
            phase(xb_ref, xa_ref)

        @pl.when(c == nsteps - 1)
        def _():
            out_ref[...] = (
                jnp.dot(h_ref[...].astype(jnp.bfloat16), wfc_ref[...],
                        preferred_element_type=jnp.float32) + bfc_ref[...]
            ).astype(out_ref.dtype)

    return body


def _fused_kernel(x_idx, embedding, w_ih_t, w_hh_t, b_rnn, w_fc_t, b_fc, ch):
    B, T = x_idx.shape
    V, H = embedding.shape
    O = w_fc_t.shape[1]
    Hp = _round_up(H, 128)
    Op = _round_up(O, 128)
    Bp = _round_up(B, 8)
    Vp = _round_up(V, 8)

    emb = _pad_to(embedding, (Vp, Hp))
    wih = _pad_to(w_ih_t, (Hp, Hp)).astype(jnp.bfloat16)
    brnn = _pad_to(b_rnn, (1, Hp))
    whh = _pad_to(w_hh_t, (Hp, Hp)).astype(jnp.bfloat16)
    wfc = _pad_to(w_fc_t, (Hp, Op)).astype(jnp.bfloat16)
    bfc = _pad_to(b_fc, (1, Op))

    n_chunks = T // ch
    body = _make_fused_body(T, ch, B, Bp, Hp)

    grid_spec = pltpu.PrefetchScalarGridSpec(
        num_scalar_prefetch=1,
        grid=(n_chunks + 1,),
        in_specs=[
            pl.BlockSpec(memory_space=pl.ANY),
            pl.BlockSpec((Hp, Hp), lambda c, idx: (0, 0)),
            pl.BlockSpec((1, Hp), lambda c, idx: (0, 0)),
            pl.BlockSpec((Hp, Hp), lambda c, idx: (0, 0)),
            pl.BlockSpec((Hp, Op), lambda c, idx: (0, 0)),
            pl.BlockSpec((1, Op), lambda c, idx: (0, 0)),
        ],
        out_specs=pl.BlockSpec((Bp, Op), lambda c, idx: (0, 0)),
        scratch_shapes=[
            pltpu.VMEM((Bp, Hp), jnp.float32),
            pltpu.VMEM((Vp, 1, Hp), jnp.float32),
            pltpu.VMEM((ch, Bp, Hp), jnp.float32),
            pltpu.VMEM((ch, Bp, Hp), jnp.float32),
            pltpu.VMEM((ch, Bp, Hp), jnp.float32),
            pltpu.SemaphoreType.DMA,
        ],
    )
    out_p = pl.pallas_call(
        body,
        out_shape=jax.ShapeDtypeStruct((Bp, Op), jnp.float32),
        grid_spec=grid_spec,
        compiler_params=pltpu.CompilerParams(
            dimension_semantics=("arbitrary",),
            vmem_limit_bytes=63 * (1 << 20),
        ),
    )(x_idx.T.reshape(-1), emb, wih, brnn, whh, wfc, bfc)
    return out_p[:B, :O]


def _make_chunked_body(total_t, chunk, tb, hp, needs_mask):
    def body(emb_ref, wih_ref, brnn_ref, whh_ref, wfc_ref, bfc_ref,
             out_ref, h_ref, xp_ref):
        c = pl.program_id(0)

        @pl.when(c == 0)
        def _():
            h_ref[...] = jnp.zeros_like(h_ref)

        wih = wih_ref[...]
        brnn = brnn_ref[...]
        for g in range(chunk):
            xp_ref[g] = (jnp.dot(emb_ref[g], wih,
                                 preferred_element_type=jnp.float32) + brnn)

        whh = whh_ref[...]
        base = c * chunk
        h = h_ref[...]
        for t in range(chunk):
            pre = xp_ref[t] + jnp.dot(h, whh,
                                      preferred_element_type=jnp.float32)
            h_new = jnp.tanh(pre)
            if needs_mask:
                h_new = jnp.where(base + t < total_t, h_new, h)
            h = h_new
        h_ref[...] = h

        @pl.when(c == pl.num_programs(0) - 1)
        def _():
            out_ref[...] = (
                jnp.dot(h, wfc_ref[...], preferred_element_type=jnp.float32)
                + bfc_ref[...]
            ).astype(out_ref.dtype)

    return body


def _chunked_kernel(x_idx, embedding, w_ih_t, w_hh_t, b_rnn, w_fc_t, b_fc):
    B, T = x_idx.shape
    H = embedding.shape[1]
    O = w_fc_t.shape[1]
    Hp = _round_up(H, 128)
    Op = _round_up(O, 128)
    Bp = _round_up(B, 8)

    chunk = 0
    for cand in range(min(T, 32), 0, -1):
        if T % cand == 0:
            chunk = cand
            break
    if chunk < 8 and T > 32:
        chunk = 32
    n_chunks = -(-T // chunk)
    Tp = n_chunks * chunk
    needs_mask = Tp != T

    emb_tb = embedding[x_idx.T].astype(jnp.bfloat16)
    emb_tb = _pad_to(emb_tb, (Tp, Bp, Hp))
    wih = _pad_to(w_ih_t, (H, Hp)).astype(jnp.bfloat16)
    wih = _pad_to(wih, (Hp, Hp))
    brnn = _pad_to(b_rnn, (1, Hp))
    whh = _pad_to(w_hh_t, (Hp, Hp))
    wfc = _pad_to(w_fc_t, (Hp, Op))
    bfc = _pad_to(b_fc, (1, Op))

    body = _make_chunked_body(T, chunk, Bp, Hp, needs_mask)
    out_p = pl.pallas_call(
        body,
        grid=(n_chunks,),
        in_specs=[
            pl.BlockSpec((chunk, Bp, Hp), lambda c: (c, 0, 0)),
            pl.BlockSpec((Hp, Hp), lambda c: (0, 0)),
            pl.BlockSpec((1, Hp), lambda c: (0, 0)),
            pl.BlockSpec((Hp, Hp), lambda c: (0, 0)),
            pl.BlockSpec((Hp, Op), lambda c: (0, 0)),
            pl.BlockSpec((1, Op), lambda c: (0, 0)),
        ],
        out_specs=pl.BlockSpec((Bp, Op), lambda c: (0, 0)),
        out_shape=jax.ShapeDtypeStruct((Bp, Op), jnp.float32),
        scratch_shapes=[
            pltpu.VMEM((Bp, Hp), jnp.float32),
            pltpu.VMEM((chunk, Bp, Hp), jnp.float32),
        ],
        compiler_params=pltpu.CompilerParams(
            dimension_semantics=("arbitrary",),
            vmem_limit_bytes=100 * (1 << 20),
        ),
    )(emb_tb, wih, brnn, whh, wfc, bfc)
    return out_p[:B, :O]


def kernel(x_idx, embedding, w_ih_t, w_hh_t, b_rnn, w_fc_t, b_fc):
    B, T = x_idx.shape
    V, H = embedding.shape

    ch = 0
    for cand in (4, 2, 1):
        if T % cand == 0:
            ch = cand
            break

    Hp = _round_up(H, 128)
    Bp = _round_up(B, 8)
    Vp = _round_up(V, 8)
    vmem_bytes = (Vp * Hp + 3 * ch * Bp * Hp + 2 * Hp * Hp + Bp * Hp) * 4
    if (ch > 0 and B * ch <= 2048 and T >= 2 * ch
            and vmem_bytes <= 58 * (1 << 20)):
        return _fused_kernel(x_idx, embedding, w_ih_t, w_hh_t, b_rnn,
                             w_fc_t, b_fc, ch)
    return _chunked_kernel(x_idx, embedding, w_ih_t, w_hh_t, b_rnn,
                           w_fc_t, b_fc)
